# baseline (speedup 1.0000x reference)
import functools

import jax
import jax.numpy as jnp
from jax import lax
from jax.experimental import pallas as pl
from jax.experimental.pallas import tpu as pltpu
from jax.experimental.pallas import tpu_sc as plsc

VOCAB_P = 500000
DIM = 64
BATCH = 4096
HIST = 200
NC = 2
NS = 16
NW = NC * NS
BBLK = BATCH // NW
HC = 2


def _pair_gather(idx_flat, table_pairs):
    mesh = plsc.VectorSubcoreMesh(core_axis_name="c", subcore_axis_name="s")
    n_chunks = HIST // HC

    @functools.partial(
        pl.kernel,
        out_type=jax.ShapeDtypeStruct((HIST, DIM, BATCH), jnp.float32),
        mesh=mesh,
        scratch_types=[
            pltpu.VMEM((HC * BBLK,), jnp.int32),
            pltpu.VMEM((HC * BBLK,), jnp.int32),
            pltpu.VMEM((HC * BBLK, 128), jnp.float32),
            pltpu.VMEM((HC, DIM, BBLK), jnp.float32),
            pltpu.SemaphoreType.DMA,
        ],
        compiler_params=pltpu.CompilerParams(needs_layout_passes=False),
    )
    def gather_kernel(idx_hbm, tab_hbm, out_hbm, idx_v, pid_v, g_v, o_v,
                      sem):
        wid = lax.axis_index("s") * NC + lax.axis_index("c")
        bbase = wid * BBLK

        iota16 = lax.iota(jnp.int32, 16)

        def chunk_body(ci, carry):
            h0 = ci * HC
            for hl in range(HC):
                pltpu.sync_copy(
                    idx_hbm.at[pl.ds((h0 + hl) * BATCH + bbase, BBLK)],
                    idx_v.at[pl.ds(hl * BBLK, BBLK)])

            def pid_body(i, c):
                v = idx_v[pl.ds(i * 16, 16)]
                pid_v[pl.ds(i * 16, 16)] = lax.shift_right_logical(v, 1)
                return c

            lax.fori_loop(0, (HC * BBLK) // 16, pid_body, 0)

            pltpu.async_copy(tab_hbm.at[pid_v], g_v, sem).wait()

            for hl in range(HC):
                for grp in range(8):
                    t0 = hl * BBLK + grp * 16
                    rows = t0 + iota16
                    par = lax.bitwise_and(idx_v[pl.ds(t0, 16)], 1)
                    col0 = par * 64

                    def d_body(d, c, rows=rows, col0=col0, hl=hl, grp=grp):
                        vals = plsc.load_gather(g_v, [rows, col0 + d])
                        o_v[hl, d, pl.ds(grp * 16, 16)] = vals
                        return c

                    lax.fori_loop(0, DIM, d_body, 0)

            for hl in range(HC):
                pltpu.sync_copy(
                    o_v.at[hl],
                    out_hbm.at[h0 + hl, :, pl.ds(bbase, BBLK)])
            return carry

        lax.fori_loop(0, n_chunks, chunk_body, 0)

    return gather_kernel(idx_flat, table_pairs)


def kernel(x, emb_weight):
    idx_flat = x.T.reshape(BATCH * HIST)
    table_pairs = emb_weight.reshape(VOCAB_P, 128)
    out3 = _pair_gather(idx_flat, table_pairs)
    return out3.transpose(2, 0, 1)
